# baseline (speedup 1.0000x reference)
import jax
import jax.numpy as jnp
from jax.experimental import pallas as pl
from jax.experimental.pallas import tpu as pltpu

_C = 8


def _logsig(x):
    return jnp.minimum(x, 0.0) - jnp.log1p(jnp.exp(-jnp.abs(x)))


def _gru_step(gi, gh, h, Dn):
    rz = jax.nn.sigmoid(gi[0:64] + gh[0:64])
    r = rz[0:Dn]
    z = rz[32:32 + Dn]
    n = jnp.tanh(gi[64:64 + Dn] + r * gh[64:64 + Dn])
    return r, z, n


def _gru_kernel(xs_ref, ns_ref, tgt_ref, lenr_ref, lent_ref,
                WihP_ref, WhhP_ref, A1h_ref, A1e_ref, b1_ref, A2_ref,
                b2_ref, A3_ref, b3_ref, T1_ref, tb1_ref, T2_ref, tb2_ref,
                hs_ref, s_ref, aux_ref, h_sc, hp_sc, gi_sc):
    c = pl.program_id(0)
    Dn, CB = tgt_ref.shape
    Bn = lenr_ref.shape[1]
    C = CB // Bn

    @pl.when(c == 0)
    def _init():
        h_sc[...] = jnp.zeros_like(h_sc)
        aux_ref[...] = jnp.zeros_like(aux_ref)

    lenr = lenr_ref[...]
    WihP = WihP_ref[...]
    WhhP = WhhP_ref[...]
    x_mat = xs_ref[0]

    gi_sc[...] = jnp.dot(WihP, x_mat, preferred_element_type=jnp.float32)

    h = h_sc[...]
    for j in range(C):
        hp_sc[:, j * Bn:(j + 1) * Bn] = h
        gi = gi_sc[:, j * Bn:(j + 1) * Bn]
        gh = jnp.dot(WhhP, h, preferred_element_type=jnp.float32)
        r, z, n = _gru_step(gi, gh, h, Dn)
        h_new = (1.0 - z) * n + z * h
        m = (c * C + j) < lenr
        h = jnp.where(m, h_new, h)
        hs_ref[0, :, j * Bn:(j + 1) * Bn] = h
    h_sc[...] = h

    hs_mat = hs_ref[0]
    q = hs_mat * tgt_ref[...]
    sa = jax.nn.sigmoid(jnp.dot(T1_ref[...], q,
                                preferred_element_type=jnp.float32)
                        + tb1_ref[...])
    s_row = jnp.dot(T2_ref[...], sa,
                    preferred_element_type=jnp.float32) + tb2_ref[...]
    s_ref[...] = s_row.reshape(C, Bn)

    hh = jnp.dot(A1h_ref[...], hp_sc[...],
                 preferred_element_type=jnp.float32)
    xe = jnp.dot(A1e_ref[...], x_mat,
                 preferred_element_type=jnp.float32)
    ne = jnp.dot(A1e_ref[...], ns_ref[0],
                 preferred_element_type=jnp.float32)
    b1 = b1_ref[...]
    z1p = jax.nn.sigmoid(hh + xe + b1)
    z1n = jax.nn.sigmoid(hh + ne + b1)
    A2 = A2_ref[...]
    b2 = b2_ref[...]
    z2p = jax.nn.sigmoid(jnp.dot(A2, z1p,
                                 preferred_element_type=jnp.float32) + b2)
    z2n = jax.nn.sigmoid(jnp.dot(A2, z1n,
                                 preferred_element_type=jnp.float32) + b2)
    A3 = A3_ref[...]
    b3 = b3_ref[...]
    plog = jnp.dot(A3, z2p, preferred_element_type=jnp.float32) + b3
    nlog = jnp.dot(A3, z2n, preferred_element_type=jnp.float32) + b3
    terms = (-_logsig(plog)) + (-_logsig(-nlog))
    lane_t = (jax.lax.broadcasted_iota(jnp.int32, (1, CB), 1) // Bn
              + c * C)
    am = (lane_t >= 1) & (lane_t < lent_ref[...])
    aux_ref[...] += jnp.where(am, terms, 0.0)


def _augru_kernel(hs_ref, s_ref, lenr_ref, VihP_ref, VhhP_ref,
                  ev_ref, h_sc, att_sc, gi_sc):
    c = pl.program_id(0)
    S, Dn, CB = hs_ref.shape
    Tn, Bn = s_ref.shape
    C = CB // Bn

    @pl.when(c == 0)
    def _init():
        h_sc[...] = jnp.zeros_like(h_sc)
        s = s_ref[...]
        trow = jax.lax.broadcasted_iota(jnp.int32, (Tn, Bn), 0)
        sm = jnp.where(trow < lenr_ref[...], s, -1e9)
        mx = jnp.max(sm, axis=0, keepdims=True)
        e = jnp.exp(sm - mx)
        att_sc[...] = e / jnp.sum(e, axis=0, keepdims=True)

    lenr = lenr_ref[...]
    VihP = VihP_ref[...]
    VhhP = VhhP_ref[...]

    h = h_sc[...]
    for sub in range(S):
        ck = c * S + sub
        hs_mat = hs_ref[sub]
        att_chunk = att_sc[pl.ds(ck * C, C), :]
        gi_sc[...] = jnp.dot(VihP, hs_mat, preferred_element_type=jnp.float32)
        for j in range(C):
            gi = gi_sc[:, j * Bn:(j + 1) * Bn]
            at = att_chunk[j:j + 1]
            gh = jnp.dot(VhhP, h, preferred_element_type=jnp.float32)
            r, z, n = _gru_step(gi, gh, h, Dn)
            z2 = at * z
            h_new = (1.0 - z2) * h + z2 * n
            m = (ck * C + j) < lenr
            h = jnp.where(m, h_new, h)
    h_sc[...] = h
    ev_ref[...] = h


def _head_kernel(tgt_ref, nsq_ref, ev_ref, aux_ref, lenr_ref,
                 D1a_ref, D1b_ref, D1c_ref, db1_ref, D2_ref, db2_ref,
                 D3_ref, db3_ref, a1_ref, a2_ref, a3_ref,
                 prob_ref, auxo_ref):
    def bn(x):
        mu = jnp.mean(x, axis=1, keepdims=True)
        var = jnp.mean((x - mu) ** 2, axis=1, keepdims=True)
        return (x - mu) / jnp.sqrt(var + 1e-5)

    def dice(x, a):
        p = jax.nn.sigmoid(bn(x))
        return p * x + (1.0 - p) * a * x

    z1 = (jnp.dot(D1a_ref[...], tgt_ref[...],
                  preferred_element_type=jnp.float32)
          + jnp.dot(D1b_ref[...], nsq_ref[...],
                    preferred_element_type=jnp.float32)
          + jnp.dot(D1c_ref[...], ev_ref[...],
                    preferred_element_type=jnp.float32)
          + db1_ref[...])
    h1 = dice(bn(z1), a1_ref[...])
    h2 = dice(bn(jnp.dot(D2_ref[...], h1,
                         preferred_element_type=jnp.float32)
                 + db2_ref[...]), a2_ref[...])
    logit = dice(bn(jnp.dot(D3_ref[...], h2,
                            preferred_element_type=jnp.float32)
                    + db3_ref[...]), a3_ref[...])
    prob_ref[...] = jax.nn.sigmoid(logit)

    den = jnp.sum((lenr_ref[...] - 1).astype(jnp.float32))
    auxo_ref[...] = (jnp.sum(aux_ref[...])
                     / jnp.maximum(den, 1.0)).reshape(1, 1)


def _gate_pad(W):
    D = W.shape[0]
    Wt = W.T
    pad = jnp.zeros((32 - D, D), jnp.float32)
    return jnp.concatenate([Wt[0:D], pad, Wt[D:2 * D], pad,
                            Wt[2 * D:3 * D], pad], axis=0)


def kernel(pos_seq, neg_seq, target_item, non_seq, seq_lengths, Wih, Whh,
           Vih, Vhh, A1, b1, A2, b2, A3, b3, T1, tb1, T2, tb2, D1, db1,
           D2, db2, D3, db3, alpha1, alpha2, alpha3):
    B, T, D = pos_seq.shape
    NS = non_seq.shape[1]
    C = _C
    nsteps = T // C
    CB = C * B

    pack = lambda a: a.reshape(B, nsteps, C, D).transpose(1, 3, 2, 0) \
                      .reshape(nsteps, D, CB)
    xs = pack(pos_seq)
    ns = pack(neg_seq)
    tgtT = target_item.T
    tgt_tiled = jnp.tile(tgtT, (1, C))
    nsqT = non_seq.T
    lenr = seq_lengths[None, :]
    lent = jnp.tile(lenr, (1, C))
    col = lambda v: v[:, None]

    seq_params = pltpu.CompilerParams(dimension_semantics=("arbitrary",))
    full = lambda shape: pl.BlockSpec(shape, lambda i: (0,) * len(shape))
    tchunk = pl.BlockSpec((1, D, CB), lambda i: (i, 0, 0))

    hs, s, aux_vec = pl.pallas_call(
        _gru_kernel,
        grid=(nsteps,),
        in_specs=[tchunk, tchunk, full((D, CB)), full((1, B)),
                  full((1, CB)), full((96, D)), full((96, D)),
                  full((32, D)), full((32, D)), full((32, 1)),
                  full((16, 32)), full((16, 1)), full((1, 16)),
                  full((1, 1)), full((40, D)), full((40, 1)),
                  full((1, 40)), full((1, 1))],
        out_specs=[tchunk,
                   pl.BlockSpec((C, B), lambda i: (i, 0)),
                   pl.BlockSpec((1, CB), lambda i: (0, 0))],
        out_shape=[jax.ShapeDtypeStruct((nsteps, D, CB), jnp.float32),
                   jax.ShapeDtypeStruct((T, B), jnp.float32),
                   jax.ShapeDtypeStruct((1, CB), jnp.float32)],
        scratch_shapes=[pltpu.VMEM((D, B), jnp.float32),
                        pltpu.VMEM((D, CB), jnp.float32),
                        pltpu.VMEM((96, CB), jnp.float32)],
        compiler_params=seq_params,
    )(xs, ns, tgt_tiled, lenr, lent, _gate_pad(Wih), _gate_pad(Whh),
      A1[:D].T, A1[D:].T, col(b1), A2.T, col(b2), A3.T, col(b3),
      T1.T, col(tb1), T2.T, col(tb2))

    S = 1
    evolution = pl.pallas_call(
        _augru_kernel,
        grid=(nsteps // S,),
        in_specs=[pl.BlockSpec((S, D, CB), lambda i: (i, 0, 0)),
                  full((T, B)), full((1, B)),
                  full((96, D)), full((96, D))],
        out_specs=pl.BlockSpec((D, B), lambda i: (0, 0)),
        out_shape=jax.ShapeDtypeStruct((D, B), jnp.float32),
        scratch_shapes=[pltpu.VMEM((D, B), jnp.float32),
                        pltpu.VMEM((T, B), jnp.float32),
                        pltpu.VMEM((96, CB), jnp.float32)],
        compiler_params=seq_params,
    )(hs, s, lenr, _gate_pad(Vih), _gate_pad(Vhh))

    prob, auxo = pl.pallas_call(
        _head_kernel,
        grid=(1,),
        in_specs=[full((D, B)), full((NS, B)), full((D, B)),
                  full((1, CB)), full((1, B)), full((64, D)),
                  full((64, NS)), full((64, D)), full((64, 1)),
                  full((16, 64)), full((16, 1)), full((1, 16)),
                  full((1, 1)), full((64, 1)), full((16, 1)),
                  full((1, 1))],
        out_specs=[pl.BlockSpec((1, B), lambda i: (0, 0)),
                   pl.BlockSpec((1, 1), lambda i: (0, 0))],
        out_shape=[jax.ShapeDtypeStruct((1, B), jnp.float32),
                   jax.ShapeDtypeStruct((1, 1), jnp.float32)],
    )(tgtT, nsqT, evolution, aux_vec, lenr, D1[:D].T, D1[D:D + NS].T,
      D1[D + NS:].T, col(db1), D2.T, col(db2), D3.T, col(db3),
      col(alpha1), col(alpha2), col(alpha3))

    return (prob.T, auxo[0, 0])


# speedup vs baseline: 1.0558x; 1.0558x over previous
import jax
import jax.numpy as jnp
from jax.experimental import pallas as pl
from jax.experimental.pallas import tpu as pltpu

_C = 8


def _logsig(x):
    return jnp.minimum(x, 0.0) - jnp.log1p(jnp.exp(-jnp.abs(x)))


def _gru_step(gi, gh, h, Dn):
    rz = jax.nn.sigmoid(gi[0:64] + gh[0:64])
    r = rz[0:Dn]
    z = rz[32:32 + Dn]
    n = jnp.tanh(gi[64:64 + Dn] + r * gh[64:64 + Dn])
    return r, z, n


def _gru_kernel(xs_ref, ns_ref, tgt_ref, lenr_ref, lent_ref,
                WihP_ref, WhhP_ref, A1h_ref, A1e_ref, b1_ref, A2_ref,
                b2_ref, A3_ref, b3_ref, T1_ref, tb1_ref, T2_ref, tb2_ref,
                hs_ref, s_ref, aux_ref, h_sc, hp_sc):
    c = pl.program_id(0)
    Dn, CB = tgt_ref.shape
    Bn = lenr_ref.shape[1]
    C = CB // Bn

    @pl.when(c == 0)
    def _init():
        h_sc[...] = jnp.zeros_like(h_sc)
        aux_ref[...] = jnp.zeros_like(aux_ref)

    lenr = lenr_ref[...]
    WihP = WihP_ref[...]
    WhhP = WhhP_ref[...]
    x_mat = xs_ref[0]

    h = h_sc[...]
    for j in range(C):
        hp_sc[:, j * Bn:(j + 1) * Bn] = h
        gi = jnp.dot(WihP, x_mat[:, j * Bn:(j + 1) * Bn],
                     preferred_element_type=jnp.float32)
        gh = jnp.dot(WhhP, h, preferred_element_type=jnp.float32)
        r, z, n = _gru_step(gi, gh, h, Dn)
        h_new = (1.0 - z) * n + z * h
        m = (c * C + j) < lenr
        h = jnp.where(m, h_new, h)
        hs_ref[0, :, j * Bn:(j + 1) * Bn] = h
    h_sc[...] = h

    hs_mat = hs_ref[0]
    q = hs_mat * tgt_ref[...]
    sa = jax.nn.sigmoid(jnp.dot(T1_ref[...], q,
                                preferred_element_type=jnp.float32)
                        + tb1_ref[...])
    s_row = jnp.dot(T2_ref[...], sa,
                    preferred_element_type=jnp.float32) + tb2_ref[...]
    s_ref[...] = s_row.reshape(C, Bn)

    hh = jnp.dot(A1h_ref[...], hp_sc[...],
                 preferred_element_type=jnp.float32)
    xe = jnp.dot(A1e_ref[...], x_mat,
                 preferred_element_type=jnp.float32)
    ne = jnp.dot(A1e_ref[...], ns_ref[0],
                 preferred_element_type=jnp.float32)
    b1 = b1_ref[...]
    z1p = jax.nn.sigmoid(hh + xe + b1)
    z1n = jax.nn.sigmoid(hh + ne + b1)
    A2 = A2_ref[...]
    b2 = b2_ref[...]
    z2p = jax.nn.sigmoid(jnp.dot(A2, z1p,
                                 preferred_element_type=jnp.float32) + b2)
    z2n = jax.nn.sigmoid(jnp.dot(A2, z1n,
                                 preferred_element_type=jnp.float32) + b2)
    A3 = A3_ref[...]
    b3 = b3_ref[...]
    plog = jnp.dot(A3, z2p, preferred_element_type=jnp.float32) + b3
    nlog = jnp.dot(A3, z2n, preferred_element_type=jnp.float32) + b3
    terms = (-_logsig(plog)) + (-_logsig(-nlog))
    lane_t = (jax.lax.broadcasted_iota(jnp.int32, (1, CB), 1) // Bn
              + c * C)
    am = (lane_t >= 1) & (lane_t < lent_ref[...])
    aux_ref[...] += jnp.where(am, terms, 0.0)


def _augru_kernel(hs_ref, s_ref, lenr_ref, VihP_ref, VhhP_ref,
                  ev_ref, h_sc, att_sc):
    c = pl.program_id(0)
    S, Dn, CB = hs_ref.shape
    Tn, Bn = s_ref.shape
    C = CB // Bn

    @pl.when(c == 0)
    def _init():
        h_sc[...] = jnp.zeros_like(h_sc)
        s = s_ref[...]
        trow = jax.lax.broadcasted_iota(jnp.int32, (Tn, Bn), 0)
        sm = jnp.where(trow < lenr_ref[...], s, -1e9)
        mx = jnp.max(sm, axis=0, keepdims=True)
        e = jnp.exp(sm - mx)
        att_sc[...] = e / jnp.sum(e, axis=0, keepdims=True)

    lenr = lenr_ref[...]
    VihP = VihP_ref[...]
    VhhP = VhhP_ref[...]

    h = h_sc[...]
    for sub in range(S):
        ck = c * S + sub
        hs_mat = hs_ref[sub]
        att_chunk = att_sc[pl.ds(ck * C, C), :]
        for j in range(C):
            gi = jnp.dot(VihP, hs_mat[:, j * Bn:(j + 1) * Bn],
                         preferred_element_type=jnp.float32)
            at = att_chunk[j:j + 1]
            gh = jnp.dot(VhhP, h, preferred_element_type=jnp.float32)
            r, z, n = _gru_step(gi, gh, h, Dn)
            z2 = at * z
            h_new = (1.0 - z2) * h + z2 * n
            m = (ck * C + j) < lenr
            h = jnp.where(m, h_new, h)
    h_sc[...] = h
    ev_ref[...] = h


def _head_kernel(tgt_ref, nsq_ref, ev_ref, aux_ref, lenr_ref,
                 D1a_ref, D1b_ref, D1c_ref, db1_ref, D2_ref, db2_ref,
                 D3_ref, db3_ref, a1_ref, a2_ref, a3_ref,
                 prob_ref, auxo_ref):
    def bn(x):
        mu = jnp.mean(x, axis=1, keepdims=True)
        var = jnp.mean((x - mu) ** 2, axis=1, keepdims=True)
        return (x - mu) / jnp.sqrt(var + 1e-5)

    def dice(x, a):
        p = jax.nn.sigmoid(bn(x))
        return p * x + (1.0 - p) * a * x

    z1 = (jnp.dot(D1a_ref[...], tgt_ref[...],
                  preferred_element_type=jnp.float32)
          + jnp.dot(D1b_ref[...], nsq_ref[...],
                    preferred_element_type=jnp.float32)
          + jnp.dot(D1c_ref[...], ev_ref[...],
                    preferred_element_type=jnp.float32)
          + db1_ref[...])
    h1 = dice(bn(z1), a1_ref[...])
    h2 = dice(bn(jnp.dot(D2_ref[...], h1,
                         preferred_element_type=jnp.float32)
                 + db2_ref[...]), a2_ref[...])
    logit = dice(bn(jnp.dot(D3_ref[...], h2,
                            preferred_element_type=jnp.float32)
                    + db3_ref[...]), a3_ref[...])
    prob_ref[...] = jax.nn.sigmoid(logit)

    den = jnp.sum((lenr_ref[...] - 1).astype(jnp.float32))
    auxo_ref[...] = (jnp.sum(aux_ref[...])
                     / jnp.maximum(den, 1.0)).reshape(1, 1)


def _gate_pad(W):
    D = W.shape[0]
    Wt = W.T
    pad = jnp.zeros((32 - D, D), jnp.float32)
    return jnp.concatenate([Wt[0:D], pad, Wt[D:2 * D], pad,
                            Wt[2 * D:3 * D], pad], axis=0)


def kernel(pos_seq, neg_seq, target_item, non_seq, seq_lengths, Wih, Whh,
           Vih, Vhh, A1, b1, A2, b2, A3, b3, T1, tb1, T2, tb2, D1, db1,
           D2, db2, D3, db3, alpha1, alpha2, alpha3):
    B, T, D = pos_seq.shape
    NS = non_seq.shape[1]
    C = _C
    nsteps = T // C
    CB = C * B

    pack = lambda a: a.reshape(B, nsteps, C, D).transpose(1, 3, 2, 0) \
                      .reshape(nsteps, D, CB)
    xs = pack(pos_seq)
    ns = pack(neg_seq)
    tgtT = target_item.T
    tgt_tiled = jnp.tile(tgtT, (1, C))
    nsqT = non_seq.T
    lenr = seq_lengths[None, :]
    lent = jnp.tile(lenr, (1, C))
    col = lambda v: v[:, None]

    seq_params = pltpu.CompilerParams(dimension_semantics=("arbitrary",))
    full = lambda shape: pl.BlockSpec(shape, lambda i: (0,) * len(shape))
    tchunk = pl.BlockSpec((1, D, CB), lambda i: (i, 0, 0))

    hs, s, aux_vec = pl.pallas_call(
        _gru_kernel,
        grid=(nsteps,),
        in_specs=[tchunk, tchunk, full((D, CB)), full((1, B)),
                  full((1, CB)), full((96, D)), full((96, D)),
                  full((32, D)), full((32, D)), full((32, 1)),
                  full((16, 32)), full((16, 1)), full((1, 16)),
                  full((1, 1)), full((40, D)), full((40, 1)),
                  full((1, 40)), full((1, 1))],
        out_specs=[tchunk,
                   pl.BlockSpec((C, B), lambda i: (i, 0)),
                   pl.BlockSpec((1, CB), lambda i: (0, 0))],
        out_shape=[jax.ShapeDtypeStruct((nsteps, D, CB), jnp.float32),
                   jax.ShapeDtypeStruct((T, B), jnp.float32),
                   jax.ShapeDtypeStruct((1, CB), jnp.float32)],
        scratch_shapes=[pltpu.VMEM((D, B), jnp.float32),
                        pltpu.VMEM((D, CB), jnp.float32)],
        compiler_params=seq_params,
    )(xs, ns, tgt_tiled, lenr, lent, _gate_pad(Wih), _gate_pad(Whh),
      A1[:D].T, A1[D:].T, col(b1), A2.T, col(b2), A3.T, col(b3),
      T1.T, col(tb1), T2.T, col(tb2))

    S = 1
    evolution = pl.pallas_call(
        _augru_kernel,
        grid=(nsteps // S,),
        in_specs=[pl.BlockSpec((S, D, CB), lambda i: (i, 0, 0)),
                  full((T, B)), full((1, B)),
                  full((96, D)), full((96, D))],
        out_specs=pl.BlockSpec((D, B), lambda i: (0, 0)),
        out_shape=jax.ShapeDtypeStruct((D, B), jnp.float32),
        scratch_shapes=[pltpu.VMEM((D, B), jnp.float32),
                        pltpu.VMEM((T, B), jnp.float32)],
        compiler_params=seq_params,
    )(hs, s, lenr, _gate_pad(Vih), _gate_pad(Vhh))

    prob, auxo = pl.pallas_call(
        _head_kernel,
        grid=(1,),
        in_specs=[full((D, B)), full((NS, B)), full((D, B)),
                  full((1, CB)), full((1, B)), full((64, D)),
                  full((64, NS)), full((64, D)), full((64, 1)),
                  full((16, 64)), full((16, 1)), full((1, 16)),
                  full((1, 1)), full((64, 1)), full((16, 1)),
                  full((1, 1))],
        out_specs=[pl.BlockSpec((1, B), lambda i: (0, 0)),
                   pl.BlockSpec((1, 1), lambda i: (0, 0))],
        out_shape=[jax.ShapeDtypeStruct((1, B), jnp.float32),
                   jax.ShapeDtypeStruct((1, 1), jnp.float32)],
    )(tgtT, nsqT, evolution, aux_vec, lenr, D1[:D].T, D1[D:D + NS].T,
      D1[D + NS:].T, col(db1), D2.T, col(db2), D3.T, col(db3),
      col(alpha1), col(alpha2), col(alpha3))

    return (prob.T, auxo[0, 0])


# speedup vs baseline: 1.0568x; 1.0009x over previous
import jax
import jax.numpy as jnp
from jax.experimental import pallas as pl
from jax.experimental.pallas import tpu as pltpu

_C = 8


def _logsig(x):
    return jnp.minimum(x, 0.0) - jnp.log1p(jnp.exp(-jnp.abs(x)))


def _gru_step(gi, gh, h, Dn):
    rz = jax.nn.sigmoid(gi[0:64] + gh[0:64])
    r = rz[0:Dn]
    z = rz[32:32 + Dn]
    n = jnp.tanh(gi[64:64 + Dn] + r * gh[64:64 + Dn])
    return r, z, n


def _gru_kernel(xs_ref, ns_ref, tgt_ref, lenr_ref, lent_ref,
                WihP_ref, WhhP_ref, A1h_ref, A1e_ref, b1_ref, A2_ref,
                b2_ref, A3_ref, b3_ref, T1_ref, tb1_ref, T2_ref, tb2_ref,
                hs_ref, s_ref, aux_ref, h_sc):
    c = pl.program_id(0)
    Dn, CB = tgt_ref.shape
    Bn = lenr_ref.shape[1]
    C = CB // Bn

    @pl.when(c == 0)
    def _init():
        h_sc[...] = jnp.zeros_like(h_sc)
        aux_ref[...] = jnp.zeros_like(aux_ref)

    lenr = lenr_ref[...]
    WihP = WihP_ref[...]
    WhhP = WhhP_ref[...]
    x_mat = xs_ref[0]

    h_start = h_sc[...]
    h = h_start
    for j in range(C):
        gi = jnp.dot(WihP, x_mat[:, j * Bn:(j + 1) * Bn],
                     preferred_element_type=jnp.float32)
        gh = jnp.dot(WhhP, h, preferred_element_type=jnp.float32)
        r, z, n = _gru_step(gi, gh, h, Dn)
        h_new = (1.0 - z) * n + z * h
        m = (c * C + j) < lenr
        h = jnp.where(m, h_new, h)
        hs_ref[0, :, j * Bn:(j + 1) * Bn] = h
    h_sc[...] = h

    hs_mat = hs_ref[0]
    q = hs_mat * tgt_ref[...]
    sa = jax.nn.sigmoid(jnp.dot(T1_ref[...], q,
                                preferred_element_type=jnp.float32)
                        + tb1_ref[...])
    s_row = jnp.dot(T2_ref[...], sa,
                    preferred_element_type=jnp.float32) + tb2_ref[...]
    s_ref[...] = s_row.reshape(C, Bn)

    hp = jnp.concatenate([h_start, hs_mat[:, :CB - Bn]], axis=1)
    hh = jnp.dot(A1h_ref[...], hp,
                 preferred_element_type=jnp.float32)
    xe = jnp.dot(A1e_ref[...], x_mat,
                 preferred_element_type=jnp.float32)
    ne = jnp.dot(A1e_ref[...], ns_ref[0],
                 preferred_element_type=jnp.float32)
    b1 = b1_ref[...]
    z1p = jax.nn.sigmoid(hh + xe + b1)
    z1n = jax.nn.sigmoid(hh + ne + b1)
    A2 = A2_ref[...]
    b2 = b2_ref[...]
    z2p = jax.nn.sigmoid(jnp.dot(A2, z1p,
                                 preferred_element_type=jnp.float32) + b2)
    z2n = jax.nn.sigmoid(jnp.dot(A2, z1n,
                                 preferred_element_type=jnp.float32) + b2)
    A3 = A3_ref[...]
    b3 = b3_ref[...]
    plog = jnp.dot(A3, z2p, preferred_element_type=jnp.float32) + b3
    nlog = jnp.dot(A3, z2n, preferred_element_type=jnp.float32) + b3
    terms = (-_logsig(plog)) + (-_logsig(-nlog))
    lane_t = (jax.lax.broadcasted_iota(jnp.int32, (1, CB), 1) // Bn
              + c * C)
    am = (lane_t >= 1) & (lane_t < lent_ref[...])
    aux_ref[...] += jnp.where(am, terms, 0.0)


def _augru_kernel(hs_ref, s_ref, lenr_ref, VihP_ref, VhhP_ref,
                  tgt_ref, nsq_ref, aux_ref,
                  D1a_ref, D1b_ref, D1c_ref, db1_ref, D2_ref, db2_ref,
                  D3_ref, db3_ref, a1_ref, a2_ref, a3_ref,
                  prob_ref, auxo_ref, h_sc, att_sc):
    c = pl.program_id(0)
    S, Dn, CB = hs_ref.shape
    Tn, Bn = s_ref.shape
    C = CB // Bn

    @pl.when(c == 0)
    def _init():
        h_sc[...] = jnp.zeros_like(h_sc)
        s = s_ref[...]
        trow = jax.lax.broadcasted_iota(jnp.int32, (Tn, Bn), 0)
        sm = jnp.where(trow < lenr_ref[...], s, -1e9)
        mx = jnp.max(sm, axis=0, keepdims=True)
        e = jnp.exp(sm - mx)
        att_sc[...] = e / jnp.sum(e, axis=0, keepdims=True)

    lenr = lenr_ref[...]
    VihP = VihP_ref[...]
    VhhP = VhhP_ref[...]

    h = h_sc[...]
    for sub in range(S):
        ck = c * S + sub
        hs_mat = hs_ref[sub]
        att_chunk = att_sc[pl.ds(ck * C, C), :]
        for j in range(C):
            gi = jnp.dot(VihP, hs_mat[:, j * Bn:(j + 1) * Bn],
                         preferred_element_type=jnp.float32)
            at = att_chunk[j:j + 1]
            gh = jnp.dot(VhhP, h, preferred_element_type=jnp.float32)
            r, z, n = _gru_step(gi, gh, h, Dn)
            z2 = at * z
            h_new = (1.0 - z2) * h + z2 * n
            m = (ck * C + j) < lenr
            h = jnp.where(m, h_new, h)
    h_sc[...] = h

    @pl.when(c == pl.num_programs(0) - 1)
    def _head():
        _head_math(h, tgt_ref, nsq_ref, aux_ref, lenr_ref, D1a_ref,
                   D1b_ref, D1c_ref, db1_ref, D2_ref, db2_ref, D3_ref,
                   db3_ref, a1_ref, a2_ref, a3_ref, prob_ref, auxo_ref)


def _head_math(ev, tgt_ref, nsq_ref, aux_ref, lenr_ref,
               D1a_ref, D1b_ref, D1c_ref, db1_ref, D2_ref, db2_ref,
               D3_ref, db3_ref, a1_ref, a2_ref, a3_ref,
               prob_ref, auxo_ref):
    def bn(x):
        mu = jnp.mean(x, axis=1, keepdims=True)
        var = jnp.mean((x - mu) ** 2, axis=1, keepdims=True)
        return (x - mu) / jnp.sqrt(var + 1e-5)

    def dice(x, a):
        p = jax.nn.sigmoid(bn(x))
        return p * x + (1.0 - p) * a * x

    z1 = (jnp.dot(D1a_ref[...], tgt_ref[...],
                  preferred_element_type=jnp.float32)
          + jnp.dot(D1b_ref[...], nsq_ref[...],
                    preferred_element_type=jnp.float32)
          + jnp.dot(D1c_ref[...], ev,
                    preferred_element_type=jnp.float32)
          + db1_ref[...])
    h1 = dice(bn(z1), a1_ref[...])
    h2 = dice(bn(jnp.dot(D2_ref[...], h1,
                         preferred_element_type=jnp.float32)
                 + db2_ref[...]), a2_ref[...])
    logit = dice(bn(jnp.dot(D3_ref[...], h2,
                            preferred_element_type=jnp.float32)
                    + db3_ref[...]), a3_ref[...])
    prob_ref[...] = jax.nn.sigmoid(logit)

    den = jnp.sum((lenr_ref[...] - 1).astype(jnp.float32))
    auxo_ref[...] = (jnp.sum(aux_ref[...])
                     / jnp.maximum(den, 1.0)).reshape(1, 1)


def _gate_pad(W):
    D = W.shape[0]
    Wt = W.T
    pad = jnp.zeros((32 - D, D), jnp.float32)
    return jnp.concatenate([Wt[0:D], pad, Wt[D:2 * D], pad,
                            Wt[2 * D:3 * D], pad], axis=0)


def kernel(pos_seq, neg_seq, target_item, non_seq, seq_lengths, Wih, Whh,
           Vih, Vhh, A1, b1, A2, b2, A3, b3, T1, tb1, T2, tb2, D1, db1,
           D2, db2, D3, db3, alpha1, alpha2, alpha3):
    B, T, D = pos_seq.shape
    NS = non_seq.shape[1]
    C = _C
    nsteps = T // C
    CB = C * B

    pack = lambda a: a.reshape(B, nsteps, C, D).transpose(1, 3, 2, 0) \
                      .reshape(nsteps, D, CB)
    xs = pack(pos_seq)
    ns = pack(neg_seq)
    tgtT = target_item.T
    tgt_tiled = jnp.tile(tgtT, (1, C))
    nsqT = non_seq.T
    lenr = seq_lengths[None, :]
    lent = jnp.tile(lenr, (1, C))
    col = lambda v: v[:, None]

    seq_params = pltpu.CompilerParams(dimension_semantics=("arbitrary",))
    full = lambda shape: pl.BlockSpec(shape, lambda i: (0,) * len(shape))
    tchunk = pl.BlockSpec((1, D, CB), lambda i: (i, 0, 0))

    hs, s, aux_vec = pl.pallas_call(
        _gru_kernel,
        grid=(nsteps,),
        in_specs=[tchunk, tchunk, full((D, CB)), full((1, B)),
                  full((1, CB)), full((96, D)), full((96, D)),
                  full((32, D)), full((32, D)), full((32, 1)),
                  full((16, 32)), full((16, 1)), full((1, 16)),
                  full((1, 1)), full((40, D)), full((40, 1)),
                  full((1, 40)), full((1, 1))],
        out_specs=[tchunk,
                   pl.BlockSpec((C, B), lambda i: (i, 0)),
                   pl.BlockSpec((1, CB), lambda i: (0, 0))],
        out_shape=[jax.ShapeDtypeStruct((nsteps, D, CB), jnp.float32),
                   jax.ShapeDtypeStruct((T, B), jnp.float32),
                   jax.ShapeDtypeStruct((1, CB), jnp.float32)],
        scratch_shapes=[pltpu.VMEM((D, B), jnp.float32)],
        compiler_params=seq_params,
    )(xs, ns, tgt_tiled, lenr, lent, _gate_pad(Wih), _gate_pad(Whh),
      A1[:D].T, A1[D:].T, col(b1), A2.T, col(b2), A3.T, col(b3),
      T1.T, col(tb1), T2.T, col(tb2))

    S = 1
    prob, auxo = pl.pallas_call(
        _augru_kernel,
        grid=(nsteps // S,),
        in_specs=[pl.BlockSpec((S, D, CB), lambda i: (i, 0, 0)),
                  full((T, B)), full((1, B)),
                  full((96, D)), full((96, D)),
                  full((D, B)), full((NS, B)), full((1, CB)),
                  full((64, D)), full((64, NS)), full((64, D)),
                  full((64, 1)), full((16, 64)), full((16, 1)),
                  full((1, 16)), full((1, 1)), full((64, 1)),
                  full((16, 1)), full((1, 1))],
        out_specs=[pl.BlockSpec((1, B), lambda i: (0, 0)),
                   pl.BlockSpec((1, 1), lambda i: (0, 0))],
        out_shape=[jax.ShapeDtypeStruct((1, B), jnp.float32),
                   jax.ShapeDtypeStruct((1, 1), jnp.float32)],
        scratch_shapes=[pltpu.VMEM((D, B), jnp.float32),
                        pltpu.VMEM((T, B), jnp.float32)],
        compiler_params=seq_params,
    )(hs, s, lenr, _gate_pad(Vih), _gate_pad(Vhh),
      tgtT, nsqT, aux_vec, D1[:D].T, D1[D:D + NS].T, D1[D + NS:].T,
      col(db1), D2.T, col(db2), D3.T, col(db3),
      col(alpha1), col(alpha2), col(alpha3))

    return (prob.T, auxo[0, 0])


# speedup vs baseline: 1.0584x; 1.0015x over previous
import jax
import jax.numpy as jnp
from jax.experimental import pallas as pl
from jax.experimental.pallas import tpu as pltpu

_C = 8


def _logsig(x):
    return jnp.minimum(x, 0.0) - jnp.log1p(jnp.exp(-jnp.abs(x)))


def _gru_step(gi, gh, h, Dn):
    rz = jax.nn.sigmoid(gi[0:64] + gh[0:64])
    r = rz[0:Dn]
    z = rz[32:32 + Dn]
    n = jnp.tanh(gi[64:64 + Dn] + r * gh[64:64 + Dn])
    return r, z, n


def _gru_kernel(xs_ref, ns_ref, tgt_ref, lenr_ref, lent_ref,
                WihP_ref, WhhP_ref, A1h_ref, A1e_ref, b1_ref, A2_ref,
                b2_ref, A3_ref, b3_ref, T1_ref, tb1_ref, T2_ref, tb2_ref,
                hs_ref, s_ref, aux_ref, h_sc):
    c = pl.program_id(0)
    Dn, CB = tgt_ref.shape
    Bn = lenr_ref.shape[1]
    C = CB // Bn

    @pl.when(c == 0)
    def _init():
        h_sc[...] = jnp.zeros_like(h_sc)
        aux_ref[...] = jnp.zeros_like(aux_ref)

    lenr = lenr_ref[...]
    WihP = WihP_ref[...]
    WhhP = WhhP_ref[...]
    x_mat = xs_ref[0]

    h_start = h_sc[...]
    h = h_start
    for j in range(C):
        gi = jnp.dot(WihP, x_mat[:, j * Bn:(j + 1) * Bn],
                     preferred_element_type=jnp.float32)
        gh = jnp.dot(WhhP, h, preferred_element_type=jnp.float32)
        r, z, n = _gru_step(gi, gh, h, Dn)
        h_new = (1.0 - z) * n + z * h
        m = (c * C + j) < lenr
        h = jnp.where(m, h_new, h)
        hs_ref[0, :, j * Bn:(j + 1) * Bn] = h
    h_sc[...] = h

    hs_mat = hs_ref[0]
    q = hs_mat * tgt_ref[...]
    sa = jax.nn.sigmoid(jnp.dot(T1_ref[...], q,
                                preferred_element_type=jnp.float32)
                        + tb1_ref[...])
    s_row = jnp.dot(T2_ref[...], sa,
                    preferred_element_type=jnp.float32) + tb2_ref[...]
    s_ref[...] = s_row.reshape(C, Bn)

    hp = jnp.concatenate([h_start, hs_mat[:, :CB - Bn]], axis=1)
    hh = jnp.dot(A1h_ref[...], hp,
                 preferred_element_type=jnp.float32)
    xe = jnp.dot(A1e_ref[...], x_mat,
                 preferred_element_type=jnp.float32)
    ne = jnp.dot(A1e_ref[...], ns_ref[0],
                 preferred_element_type=jnp.float32)
    b1 = b1_ref[...]
    z1p = jax.nn.sigmoid(hh + xe + b1)
    z1n = jax.nn.sigmoid(hh + ne + b1)
    A2 = A2_ref[...]
    b2 = b2_ref[...]
    z2p = jax.nn.sigmoid(jnp.dot(A2, z1p,
                                 preferred_element_type=jnp.float32) + b2)
    z2n = jax.nn.sigmoid(jnp.dot(A2, z1n,
                                 preferred_element_type=jnp.float32) + b2)
    A3 = A3_ref[...]
    b3 = b3_ref[...]
    plog = jnp.dot(A3, z2p, preferred_element_type=jnp.float32) + b3
    nlog = jnp.dot(A3, z2n, preferred_element_type=jnp.float32) + b3
    terms = (-_logsig(plog)) + (-_logsig(-nlog))
    lane_t = (jax.lax.broadcasted_iota(jnp.int32, (1, CB), 1) // Bn
              + c * C)
    am = (lane_t >= 1) & (lane_t < lent_ref[...])
    aux_ref[...] += jnp.where(am, terms, 0.0)


def _augru_kernel(hs_ref, s_ref, lenr_ref, VihP_ref, VhhP_ref,
                  tgt_ref, nsq_ref, aux_ref,
                  D1a_ref, D1b_ref, D1c_ref, db1_ref, D2_ref, db2_ref,
                  D3_ref, db3_ref, a1_ref, a2_ref, a3_ref,
                  prob_ref, auxo_ref, h_sc, att_sc):
    c = pl.program_id(0)
    S, Dn, CB = hs_ref.shape
    Tn, Bn = s_ref.shape
    C = CB // Bn

    @pl.when(c == 0)
    def _init():
        h_sc[...] = jnp.zeros_like(h_sc)
        s = s_ref[...]
        trow = jax.lax.broadcasted_iota(jnp.int32, (Tn, Bn), 0)
        sm = jnp.where(trow < lenr_ref[...], s, -1e9)
        mx = jnp.max(sm, axis=0, keepdims=True)
        e = jnp.exp(sm - mx)
        att_sc[...] = e / jnp.sum(e, axis=0, keepdims=True)

    lenr = lenr_ref[...]
    VihP = VihP_ref[...]
    VhhP = VhhP_ref[...]

    h = h_sc[...]
    for sub in range(S):
        ck = c * S + sub
        hs_mat = hs_ref[sub]
        att_chunk = att_sc[pl.ds(ck * C, C), :]
        for j in range(C):
            gi = jnp.dot(VihP, hs_mat[:, j * Bn:(j + 1) * Bn],
                         preferred_element_type=jnp.float32)
            at = att_chunk[j:j + 1]
            gh = jnp.dot(VhhP, h, preferred_element_type=jnp.float32)
            r, z, n = _gru_step(gi, gh, h, Dn)
            z2 = at * z
            h_new = (1.0 - z2) * h + z2 * n
            m = (ck * C + j) < lenr
            h = jnp.where(m, h_new, h)
    h_sc[...] = h

    @pl.when(c == pl.num_programs(0) - 1)
    def _head():
        _head_math(h, tgt_ref, nsq_ref, aux_ref, lenr_ref, D1a_ref,
                   D1b_ref, D1c_ref, db1_ref, D2_ref, db2_ref, D3_ref,
                   db3_ref, a1_ref, a2_ref, a3_ref, prob_ref, auxo_ref)


def _head_math(ev, tgt_ref, nsq_ref, aux_ref, lenr_ref,
               D1a_ref, D1b_ref, D1c_ref, db1_ref, D2_ref, db2_ref,
               D3_ref, db3_ref, a1_ref, a2_ref, a3_ref,
               prob_ref, auxo_ref):
    def bn(x):
        mu = jnp.mean(x, axis=1, keepdims=True)
        var = jnp.mean((x - mu) ** 2, axis=1, keepdims=True)
        return (x - mu) / jnp.sqrt(var + 1e-5)

    def dice(x, a):
        p = jax.nn.sigmoid(bn(x))
        return p * x + (1.0 - p) * a * x

    z1 = (jnp.dot(D1a_ref[...], tgt_ref[...],
                  preferred_element_type=jnp.float32)
          + jnp.dot(D1b_ref[...], nsq_ref[...],
                    preferred_element_type=jnp.float32)
          + jnp.dot(D1c_ref[...], ev,
                    preferred_element_type=jnp.float32)
          + db1_ref[...])
    h1 = dice(bn(z1), a1_ref[...])
    h2 = dice(bn(jnp.dot(D2_ref[...], h1,
                         preferred_element_type=jnp.float32)
                 + db2_ref[...]), a2_ref[...])
    logit = dice(bn(jnp.dot(D3_ref[...], h2,
                            preferred_element_type=jnp.float32)
                    + db3_ref[...]), a3_ref[...])
    prob_ref[...] = jax.nn.sigmoid(logit)

    den = jnp.sum((lenr_ref[...] - 1).astype(jnp.float32))
    auxo_ref[...] = (jnp.sum(aux_ref[...])
                     / jnp.maximum(den, 1.0)).reshape(1, 1)


def _gate_pad(W):
    D = W.shape[0]
    Wt = W.T
    pad = jnp.zeros((32 - D, D), jnp.float32)
    return jnp.concatenate([Wt[0:D], pad, Wt[D:2 * D], pad,
                            Wt[2 * D:3 * D], pad], axis=0)


def kernel(pos_seq, neg_seq, target_item, non_seq, seq_lengths, Wih, Whh,
           Vih, Vhh, A1, b1, A2, b2, A3, b3, T1, tb1, T2, tb2, D1, db1,
           D2, db2, D3, db3, alpha1, alpha2, alpha3):
    B, T, D = pos_seq.shape
    NS = non_seq.shape[1]
    C = _C
    nsteps = T // C
    CB = C * B

    pack = lambda a: a.reshape(B, nsteps, C, D).transpose(1, 3, 2, 0) \
                      .reshape(nsteps, D, CB)
    xs = pack(pos_seq)
    ns = pack(neg_seq)
    tgtT = target_item.T
    tgt_tiled = jnp.tile(tgtT, (1, C))
    nsqT = non_seq.T
    lenr = seq_lengths[None, :]
    lent = jnp.tile(lenr, (1, C))
    col = lambda v: v[:, None]

    seq_params = pltpu.CompilerParams(dimension_semantics=("arbitrary",))
    full = lambda shape: pl.BlockSpec(shape, lambda i: (0,) * len(shape))
    tchunk = pl.BlockSpec((1, D, CB), lambda i: (i, 0, 0))

    hs, s, aux_vec = pl.pallas_call(
        _gru_kernel,
        grid=(nsteps,),
        in_specs=[tchunk, tchunk, full((D, CB)), full((1, B)),
                  full((1, CB)), full((96, D)), full((96, D)),
                  full((32, D)), full((32, D)), full((32, 1)),
                  full((16, 32)), full((16, 1)), full((1, 16)),
                  full((1, 1)), full((40, D)), full((40, 1)),
                  full((1, 40)), full((1, 1))],
        out_specs=[tchunk,
                   pl.BlockSpec((C, B), lambda i: (i, 0)),
                   pl.BlockSpec((1, CB), lambda i: (0, 0))],
        out_shape=[jax.ShapeDtypeStruct((nsteps, D, CB), jnp.float32),
                   jax.ShapeDtypeStruct((T, B), jnp.float32),
                   jax.ShapeDtypeStruct((1, CB), jnp.float32)],
        scratch_shapes=[pltpu.VMEM((D, B), jnp.float32)],
        compiler_params=seq_params,
    )(xs, ns, tgt_tiled, lenr, lent, _gate_pad(Wih), _gate_pad(Whh),
      A1[:D].T, A1[D:].T, col(b1), A2.T, col(b2), A3.T, col(b3),
      T1.T, col(tb1), T2.T, col(tb2))

    S = 5 if nsteps % 5 == 0 else 1
    prob, auxo = pl.pallas_call(
        _augru_kernel,
        grid=(nsteps // S,),
        in_specs=[pl.BlockSpec((S, D, CB), lambda i: (i, 0, 0)),
                  full((T, B)), full((1, B)),
                  full((96, D)), full((96, D)),
                  full((D, B)), full((NS, B)), full((1, CB)),
                  full((64, D)), full((64, NS)), full((64, D)),
                  full((64, 1)), full((16, 64)), full((16, 1)),
                  full((1, 16)), full((1, 1)), full((64, 1)),
                  full((16, 1)), full((1, 1))],
        out_specs=[pl.BlockSpec((1, B), lambda i: (0, 0)),
                   pl.BlockSpec((1, 1), lambda i: (0, 0))],
        out_shape=[jax.ShapeDtypeStruct((1, B), jnp.float32),
                   jax.ShapeDtypeStruct((1, 1), jnp.float32)],
        scratch_shapes=[pltpu.VMEM((D, B), jnp.float32),
                        pltpu.VMEM((T, B), jnp.float32)],
        compiler_params=seq_params,
    )(hs, s, lenr, _gate_pad(Vih), _gate_pad(Vhh),
      tgtT, nsqT, aux_vec, D1[:D].T, D1[D:D + NS].T, D1[D + NS:].T,
      col(db1), D2.T, col(db2), D3.T, col(db3),
      col(alpha1), col(alpha2), col(alpha3))

    return (prob.T, auxo[0, 0])


# speedup vs baseline: 1.1258x; 1.0637x over previous
import jax
import jax.numpy as jnp
from jax.experimental import pallas as pl
from jax.experimental.pallas import tpu as pltpu

_C = 8


def _logsig(x):
    return jnp.minimum(x, 0.0) - jnp.log1p(jnp.exp(-jnp.abs(x)))


def _gru_step(gi, gh, h, Dn):
    rz = jax.nn.sigmoid(gi[0:64] + gh[0:64])
    r = rz[0:Dn]
    z = rz[32:32 + Dn]
    n = jnp.tanh(gi[64:64 + Dn] + r * gh[64:64 + Dn])
    return r, z, n


def _gru_kernel(xs_ref, ns_ref, tgt_ref, lenr_ref, lent_ref,
                WihP_ref, WhhP_ref, A1h_ref, A1e_ref, b1_ref, A2_ref,
                b2_ref, A3_ref, b3_ref, T1_ref, tb1_ref, T2_ref, tb2_ref,
                hs_ref, s_ref, aux_ref, h_sc):
    c = pl.program_id(0)
    Dn, CB = tgt_ref.shape
    Bn = lenr_ref.shape[1]
    C = CB // Bn

    @pl.when(c == 0)
    def _init():
        h_sc[...] = jnp.zeros_like(h_sc)
        aux_ref[...] = jnp.zeros_like(aux_ref)

    lenr = lenr_ref[...]
    WihP = WihP_ref[...]
    WhhP = WhhP_ref[...]
    x_mat = xs_ref[0]

    h_start = h_sc[...]
    h = h_start
    for j in range(C):
        gi = jnp.dot(WihP, x_mat[:, j * Bn:(j + 1) * Bn],
                     preferred_element_type=jnp.float32)
        gh = jnp.dot(WhhP, h, preferred_element_type=jnp.float32)
        r, z, n = _gru_step(gi, gh, h, Dn)
        h_new = (1.0 - z) * n + z * h
        m = (c * C + j) < lenr
        h = jnp.where(m, h_new, h)
        hs_ref[0, :, j * Bn:(j + 1) * Bn] = h.astype(hs_ref.dtype)
    h_sc[...] = h

    hs_mat = hs_ref[0]
    q = hs_mat * tgt_ref[...]
    sa = jax.nn.sigmoid(jnp.dot(T1_ref[...], q,
                                preferred_element_type=jnp.float32)
                        + tb1_ref[...])
    s_row = jnp.dot(T2_ref[...], sa,
                    preferred_element_type=jnp.float32) + tb2_ref[...]
    s_ref[...] = s_row.reshape(C, Bn)

    hp = jnp.concatenate([h_start.astype(hs_mat.dtype),
                          hs_mat[:, :CB - Bn]], axis=1)
    hh = jnp.dot(A1h_ref[...], hp,
                 preferred_element_type=jnp.float32)
    xe = jnp.dot(A1e_ref[...], x_mat,
                 preferred_element_type=jnp.float32)
    ne = jnp.dot(A1e_ref[...], ns_ref[0],
                 preferred_element_type=jnp.float32)
    b1 = b1_ref[...]
    z1p = jax.nn.sigmoid(hh + xe + b1)
    z1n = jax.nn.sigmoid(hh + ne + b1)
    A2 = A2_ref[...]
    b2 = b2_ref[...]
    z2p = jax.nn.sigmoid(jnp.dot(A2, z1p,
                                 preferred_element_type=jnp.float32) + b2)
    z2n = jax.nn.sigmoid(jnp.dot(A2, z1n,
                                 preferred_element_type=jnp.float32) + b2)
    A3 = A3_ref[...]
    b3 = b3_ref[...]
    plog = jnp.dot(A3, z2p, preferred_element_type=jnp.float32) + b3
    nlog = jnp.dot(A3, z2n, preferred_element_type=jnp.float32) + b3
    terms = (-_logsig(plog)) + (-_logsig(-nlog))
    lane_t = (jax.lax.broadcasted_iota(jnp.int32, (1, CB), 1) // Bn
              + c * C)
    am = (lane_t >= 1) & (lane_t < lent_ref[...])
    aux_ref[...] += jnp.where(am, terms, 0.0)


def _augru_kernel(hs_ref, s_ref, lenr_ref, VihP_ref, VhhP_ref,
                  tgt_ref, nsq_ref, aux_ref,
                  D1a_ref, D1b_ref, D1c_ref, db1_ref, D2_ref, db2_ref,
                  D3_ref, db3_ref, a1_ref, a2_ref, a3_ref,
                  prob_ref, auxo_ref, h_sc, att_sc):
    c = pl.program_id(0)
    S, Dn, CB = hs_ref.shape
    Tn, Bn = s_ref.shape
    C = CB // Bn

    @pl.when(c == 0)
    def _init():
        h_sc[...] = jnp.zeros_like(h_sc)
        s = s_ref[...]
        trow = jax.lax.broadcasted_iota(jnp.int32, (Tn, Bn), 0)
        sm = jnp.where(trow < lenr_ref[...], s, -1e9)
        mx = jnp.max(sm, axis=0, keepdims=True)
        e = jnp.exp(sm - mx)
        att_sc[...] = e / jnp.sum(e, axis=0, keepdims=True)

    lenr = lenr_ref[...]
    VihP = VihP_ref[...]
    VhhP = VhhP_ref[...]

    h = h_sc[...]
    for sub in range(S):
        ck = c * S + sub
        hs_mat = hs_ref[sub]
        att_chunk = att_sc[pl.ds(ck * C, C), :]
        for j in range(C):
            gi = jnp.dot(VihP, hs_mat[:, j * Bn:(j + 1) * Bn],
                         preferred_element_type=jnp.float32)
            at = att_chunk[j:j + 1]
            gh = jnp.dot(VhhP, h, preferred_element_type=jnp.float32)
            r, z, n = _gru_step(gi, gh, h, Dn)
            z2 = at * z
            h_new = (1.0 - z2) * h + z2 * n
            m = (ck * C + j) < lenr
            h = jnp.where(m, h_new, h)
    h_sc[...] = h

    @pl.when(c == pl.num_programs(0) - 1)
    def _head():
        _head_math(h, tgt_ref, nsq_ref, aux_ref, lenr_ref, D1a_ref,
                   D1b_ref, D1c_ref, db1_ref, D2_ref, db2_ref, D3_ref,
                   db3_ref, a1_ref, a2_ref, a3_ref, prob_ref, auxo_ref)


def _head_math(ev, tgt_ref, nsq_ref, aux_ref, lenr_ref,
               D1a_ref, D1b_ref, D1c_ref, db1_ref, D2_ref, db2_ref,
               D3_ref, db3_ref, a1_ref, a2_ref, a3_ref,
               prob_ref, auxo_ref):
    def bn(x):
        mu = jnp.mean(x, axis=1, keepdims=True)
        var = jnp.mean((x - mu) ** 2, axis=1, keepdims=True)
        return (x - mu) / jnp.sqrt(var + 1e-5)

    def dice(x, a):
        p = jax.nn.sigmoid(bn(x))
        return p * x + (1.0 - p) * a * x

    z1 = (jnp.dot(D1a_ref[...], tgt_ref[...],
                  preferred_element_type=jnp.float32)
          + jnp.dot(D1b_ref[...], nsq_ref[...],
                    preferred_element_type=jnp.float32)
          + jnp.dot(D1c_ref[...], ev,
                    preferred_element_type=jnp.float32)
          + db1_ref[...])
    h1 = dice(bn(z1), a1_ref[...])
    h2 = dice(bn(jnp.dot(D2_ref[...], h1,
                         preferred_element_type=jnp.float32)
                 + db2_ref[...]), a2_ref[...])
    logit = dice(bn(jnp.dot(D3_ref[...], h2,
                            preferred_element_type=jnp.float32)
                    + db3_ref[...]), a3_ref[...])
    prob_ref[...] = jax.nn.sigmoid(logit)

    den = jnp.sum((lenr_ref[...] - 1).astype(jnp.float32))
    auxo_ref[...] = (jnp.sum(aux_ref[...])
                     / jnp.maximum(den, 1.0)).reshape(1, 1)


def _gate_pad(W):
    D = W.shape[0]
    Wt = W.T
    pad = jnp.zeros((32 - D, D), jnp.float32)
    return jnp.concatenate([Wt[0:D], pad, Wt[D:2 * D], pad,
                            Wt[2 * D:3 * D], pad], axis=0)


def kernel(pos_seq, neg_seq, target_item, non_seq, seq_lengths, Wih, Whh,
           Vih, Vhh, A1, b1, A2, b2, A3, b3, T1, tb1, T2, tb2, D1, db1,
           D2, db2, D3, db3, alpha1, alpha2, alpha3):
    B, T, D = pos_seq.shape
    NS = non_seq.shape[1]
    C = _C
    nsteps = T // C
    CB = C * B

    pack = lambda a: a.astype(jnp.bfloat16) \
                      .reshape(B, nsteps, C, D).transpose(1, 3, 2, 0) \
                      .reshape(nsteps, D, CB)
    xs = pack(pos_seq)
    ns = pack(neg_seq)
    tgtT = target_item.T
    tgt_tiled = jnp.tile(tgtT, (1, C))
    nsqT = non_seq.T
    lenr = seq_lengths[None, :]
    lent = jnp.tile(lenr, (1, C))
    col = lambda v: v[:, None]

    seq_params = pltpu.CompilerParams(dimension_semantics=("arbitrary",))
    full = lambda shape: pl.BlockSpec(shape, lambda i: (0,) * len(shape))
    tchunk = pl.BlockSpec((1, D, CB), lambda i: (i, 0, 0))

    hs, s, aux_vec = pl.pallas_call(
        _gru_kernel,
        grid=(nsteps,),
        in_specs=[tchunk, tchunk, full((D, CB)), full((1, B)),
                  full((1, CB)), full((96, D)), full((96, D)),
                  full((32, D)), full((32, D)), full((32, 1)),
                  full((16, 32)), full((16, 1)), full((1, 16)),
                  full((1, 1)), full((40, D)), full((40, 1)),
                  full((1, 40)), full((1, 1))],
        out_specs=[tchunk,
                   pl.BlockSpec((C, B), lambda i: (i, 0)),
                   pl.BlockSpec((1, CB), lambda i: (0, 0))],
        out_shape=[jax.ShapeDtypeStruct((nsteps, D, CB), jnp.bfloat16),
                   jax.ShapeDtypeStruct((T, B), jnp.float32),
                   jax.ShapeDtypeStruct((1, CB), jnp.float32)],
        scratch_shapes=[pltpu.VMEM((D, B), jnp.float32)],
        compiler_params=seq_params,
    )(xs, ns, tgt_tiled, lenr, lent,
      _gate_pad(Wih).astype(jnp.bfloat16), _gate_pad(Whh),
      A1[:D].T.astype(jnp.bfloat16), A1[D:].T.astype(jnp.bfloat16),
      col(b1), A2.T, col(b2), A3.T, col(b3),
      T1.T, col(tb1), T2.T, col(tb2))

    S = 5 if nsteps % 5 == 0 else 1
    prob, auxo = pl.pallas_call(
        _augru_kernel,
        grid=(nsteps // S,),
        in_specs=[pl.BlockSpec((S, D, CB), lambda i: (i, 0, 0)),
                  full((T, B)), full((1, B)),
                  full((96, D)), full((96, D)),
                  full((D, B)), full((NS, B)), full((1, CB)),
                  full((64, D)), full((64, NS)), full((64, D)),
                  full((64, 1)), full((16, 64)), full((16, 1)),
                  full((1, 16)), full((1, 1)), full((64, 1)),
                  full((16, 1)), full((1, 1))],
        out_specs=[pl.BlockSpec((1, B), lambda i: (0, 0)),
                   pl.BlockSpec((1, 1), lambda i: (0, 0))],
        out_shape=[jax.ShapeDtypeStruct((1, B), jnp.float32),
                   jax.ShapeDtypeStruct((1, 1), jnp.float32)],
        scratch_shapes=[pltpu.VMEM((D, B), jnp.float32),
                        pltpu.VMEM((T, B), jnp.float32)],
        compiler_params=seq_params,
    )(hs, s, lenr, _gate_pad(Vih).astype(jnp.bfloat16), _gate_pad(Vhh),
      tgtT, nsqT, aux_vec, D1[:D].T, D1[D:D + NS].T, D1[D + NS:].T,
      col(db1), D2.T, col(db2), D3.T, col(db3),
      col(alpha1), col(alpha2), col(alpha3))

    return (prob.T, auxo[0, 0])
